# baseline (speedup 1.0000x reference)
import functools

import jax
import jax.numpy as jnp
from jax import lax
from jax.experimental import pallas as pl
from jax.experimental.pallas import tpu as pltpu
from jax.experimental.pallas import tpu_sc as plsc

NT = 32
CH = 96
G = 64
BN = 10000

_sc_params = pltpu.CompilerParams(needs_layout_passes=False)
_mesh = plsc.VectorSubcoreMesh(core_axis_name="c", subcore_axis_name="s")


def _stripe_chunks(stripe, maxsz=CH):
    sizes = [maxsz] * (stripe // maxsz)
    if stripe % maxsz:
        sizes.append(stripe % maxsz)
    return sizes


def _make_deg_kernel(NCH, NA):
    @functools.partial(
        pl.kernel,
        out_type=jax.ShapeDtypeStruct((NT * NA,), jnp.float32),
        mesh=_mesh,
        scratch_types=[
            pltpu.VMEM((NCH, CH), jnp.int32),
            pltpu.VMEM((NA,), jnp.float32),
        ],
        compiler_params=_sc_params,
    )
    def deg_kernel(dst_hbm, out_hbm, dstv, degv):
        c = lax.axis_index("c")
        s = lax.axis_index("s")
        w = c * 16 + s
        pltpu.sync_copy(dst_hbm.at[w], dstv)

        zeros16 = jnp.zeros((16,), jnp.float32)

        def zero_body(i, carry):
            degv[pl.ds(i * 16, 16)] = zeros16
            return carry

        lax.fori_loop(0, NA // 16, zero_body, 0)

        ones16 = jnp.ones((16,), jnp.float32)

        def hist_body(r, carry):
            for k in range(CH // 16):
                idx = dstv[r, pl.ds(k * 16, 16)]
                plsc.addupdate_scatter(degv, [idx], ones16)
            return carry

        lax.fori_loop(0, NCH, hist_body, 0)
        pltpu.sync_copy(degv, out_hbm.at[pl.ds(w * NA, NA)])

    return deg_kernel


def _make_edge_kernel(N, NCH, NA, D):
    EP_T = NCH * CH
    stripe = NA // 16

    @functools.partial(
        pl.kernel,
        out_type=jax.ShapeDtypeStruct((2, NA, D), jnp.float32),
        mesh=_mesh,
        scratch_types=[
            pltpu.VMEM((EP_T,), jnp.int32),
            pltpu.VMEM((NCH, CH), jnp.int32),
            pltpu.VMEM((2, CH, D), jnp.float32),
            pltpu.VMEM_SHARED((NA, D), jnp.float32),
            pltpu.SemaphoreType.DMA,
            pltpu.SemaphoreType.DMA,
            pltpu.SemaphoreType.DMA,
            pltpu.SemaphoreType.DMA,
        ],
        compiler_params=_sc_params,
    )
    def edge_kernel(y_hbm, src_hbm, dst_hbm, out_hbm, srcv, dstv, buf, acc,
                    sem0, sem1, wsem0, wsem1):
        c = lax.axis_index("c")
        s = lax.axis_index("s")
        w = c * 16 + s
        pltpu.sync_copy(src_hbm.at[w], srcv)
        pltpu.sync_copy(dst_hbm.at[w], dstv)

        sems = (sem0, sem1)

        def gather(j, slot, sem):
            return pltpu.async_copy(
                y_hbm.at[srcv.at[pl.ds(j * CH, CH)]], buf.at[slot], sem)

        gather(0, 0, sem0)

        zeros16 = jnp.zeros((16,), jnp.float32)

        def zero_body(i, carry):
            r = i // (D // 16)
            k = lax.rem(i, D // 16)
            buf[1, r, pl.ds(k * 16, 16)] = zeros16
            return carry

        lax.fori_loop(0, CH * D // 16, zero_body, 0)

        base = s * stripe
        off = 0
        for sz in _stripe_chunks(stripe):
            pltpu.sync_copy(buf.at[1, pl.ds(0, sz)],
                            acc.at[pl.ds(base + off, sz)])
            off += sz
        plsc.subcore_barrier()

        def body(jj, carry):
            for b in range(2):
                j = jj * 2 + b
                nslot = 1 - b

                @pl.when(j + 1 < NCH)
                def _():
                    gather(j + 1, nslot, sems[nslot])

                pltpu.make_async_copy(
                    y_hbm.at[srcv.at[pl.ds(j * CH, CH)]], buf.at[b],
                    sems[b]).wait()
                pltpu.sync_copy(buf.at[b], acc.at[dstv.at[j]], add=True)
            return carry

        lax.fori_loop(0, NCH // 2, body, 0)
        if NCH % 2:
            j = NCH - 1
            pltpu.make_async_copy(
                y_hbm.at[srcv.at[pl.ds(j * CH, CH)]], buf.at[0],
                sem0).wait()
            pltpu.sync_copy(buf.at[0], acc.at[dstv.at[j]], add=True)
        plsc.subcore_barrier()

        wsems = (wsem0, wsem1)
        chunks = []
        off = 0
        for sz in _stripe_chunks(stripe):
            chunks.append((off, sz))
            off += sz
        for k, (off, sz) in enumerate(chunks):
            slot = k % 2
            if k >= 2:
                poff, psz = chunks[k - 2]
                pltpu.make_async_copy(
                    buf.at[slot, pl.ds(0, psz)],
                    out_hbm.at[c, pl.ds(base + poff, psz)],
                    wsems[slot]).wait()
            pltpu.sync_copy(acc.at[pl.ds(base + off, sz)],
                            buf.at[slot, pl.ds(0, sz)])
            pltpu.async_copy(buf.at[slot, pl.ds(0, sz)],
                             out_hbm.at[c, pl.ds(base + off, sz)],
                             wsems[slot])
        for k in range(max(0, len(chunks) - 2), len(chunks)):
            off, sz = chunks[k]
            pltpu.make_async_copy(
                buf.at[k % 2, pl.ds(0, sz)],
                out_hbm.at[c, pl.ds(base + off, sz)],
                wsems[k % 2]).wait()

    return edge_kernel


def _matmul1(x, W1, NB):
    N, D = x.shape
    H = W1.shape[1]

    def body(x_ref, w_ref, xw_ref):
        xw_ref[...] = jnp.dot(x_ref[...], w_ref[...],
                              preferred_element_type=jnp.float32)

    return pl.pallas_call(
        body,
        grid=(NB,),
        in_specs=[
            pl.BlockSpec((BN, D), lambda i: (i, 0)),
            pl.BlockSpec((D, H), lambda i: (0, 0)),
        ],
        out_specs=pl.BlockSpec((BN, H), lambda i: (i, 0)),
        out_shape=jax.ShapeDtypeStruct((N, H), jnp.float32),
    )(x, W1)


def _scale1(xw, deg_t, NB):
    N, H = xw.shape

    def body(xw_ref, degt_ref, y_ref, dis_ref):
        deg = jnp.sum(degt_ref[...], axis=1, keepdims=True) + 1.0
        dis = lax.rsqrt(deg)
        y_ref[...] = xw_ref[...] * dis
        dis_ref[...] = dis

    return pl.pallas_call(
        body,
        grid=(NB,),
        in_specs=[
            pl.BlockSpec((BN, H), lambda i: (i, 0)),
            pl.BlockSpec((BN, NT), lambda i: (i, 0)),
        ],
        out_specs=[
            pl.BlockSpec((BN, H), lambda i: (i, 0)),
            pl.BlockSpec((BN, 1), lambda i: (i, 0)),
        ],
        out_shape=[
            jax.ShapeDtypeStruct((N, H), jnp.float32),
            jax.ShapeDtypeStruct((N, 1), jnp.float32),
        ],
    )(xw, deg_t)


def _finish_matmul2(parts, y1, dis, b1, W2, NB):
    N, H = y1.shape

    def body(p_ref, y_ref, dis_ref, b_ref, w_ref, out_ref):
        pre = p_ref[0] + p_ref[1] + y_ref[...]
        h = jnp.maximum(dis_ref[...] * pre + b_ref[...], 0.0)
        out_ref[...] = dis_ref[...] * jnp.dot(
            h, w_ref[...], preferred_element_type=jnp.float32)

    return pl.pallas_call(
        body,
        grid=(NB,),
        in_specs=[
            pl.BlockSpec((2, BN, H), lambda i: (0, i, 0)),
            pl.BlockSpec((BN, H), lambda i: (i, 0)),
            pl.BlockSpec((BN, 1), lambda i: (i, 0)),
            pl.BlockSpec((1, H), lambda i: (0, 0)),
            pl.BlockSpec((H, H), lambda i: (0, 0)),
        ],
        out_specs=pl.BlockSpec((BN, H), lambda i: (i, 0)),
        out_shape=jax.ShapeDtypeStruct((N, H), jnp.float32),
    )(parts, y1, dis, b1, W2)


def _finish_pool_classify(parts, y2, dis, b2, batch3, Wc, bc, NB):
    N, H = y2.shape
    C = Wc.shape[1]

    def body(p_ref, y_ref, dis_ref, b_ref, bat_ref, wc_ref, bc_ref, out_ref,
             pooled, cnt):
        i = pl.program_id(0)

        @pl.when(i == 0)
        def _():
            pooled[...] = jnp.zeros_like(pooled)
            cnt[...] = jnp.zeros_like(cnt)

        pre = p_ref[0] + p_ref[1] + y_ref[...]
        h = jnp.maximum(dis_ref[...] * pre + b_ref[...], 0.0)
        bat = bat_ref[0]
        gid = lax.broadcasted_iota(jnp.int32, (G, BN), 0)
        oh = (bat == gid).astype(jnp.float32)
        pooled[...] += jnp.dot(oh, h, preferred_element_type=jnp.float32)
        cnt[...] += jnp.broadcast_to(
            jnp.sum(oh, axis=1, keepdims=True), cnt.shape)

        @pl.when(i == NB - 1)
        def _():
            mean = pooled[...] / jnp.maximum(cnt[...], 1.0)
            out_ref[...] = jnp.dot(
                mean, wc_ref[...], preferred_element_type=jnp.float32) \
                + bc_ref[...]

    return pl.pallas_call(
        body,
        grid=(NB,),
        in_specs=[
            pl.BlockSpec((2, BN, H), lambda i: (0, i, 0)),
            pl.BlockSpec((BN, H), lambda i: (i, 0)),
            pl.BlockSpec((BN, 1), lambda i: (i, 0)),
            pl.BlockSpec((1, H), lambda i: (0, 0)),
            pl.BlockSpec((1, 1, BN), lambda i: (i, 0, 0)),
            pl.BlockSpec((H, C), lambda i: (0, 0)),
            pl.BlockSpec((1, C), lambda i: (0, 0)),
        ],
        out_specs=pl.BlockSpec((G, C), lambda i: (0, 0)),
        out_shape=jax.ShapeDtypeStruct((G, C), jnp.float32),
        scratch_shapes=[
            pltpu.VMEM((G, H), jnp.float32),
            pltpu.VMEM((G, H), jnp.float32),
        ],
    )(parts, y2, dis, b2, batch3, Wc, bc)


def kernel(x, edge_index, batch, W1, b1, W2, b2, Wc, bc):
    N, D = x.shape
    H = W1.shape[1]
    E = edge_index.shape[1]
    src, dst = edge_index[0], edge_index[1]

    NCH = -(-E // (NT * CH))
    NCH += NCH % 2
    EP = NT * NCH * CH
    NA = -(-(N + 16) // 128) * 128
    NB = N // BN

    npad = EP - E
    pad_src = (jnp.arange(npad, dtype=jnp.int32) * 97) % N
    pad_dst = N + jnp.arange(npad, dtype=jnp.int32) % 16
    srcp = jnp.concatenate([src, pad_src]).reshape(NT, NCH * CH)
    dstp = jnp.concatenate([dst, pad_dst]).reshape(NT, NCH, CH)
    batch3 = batch.reshape(NB, 1, BN)

    deg_kernel = _make_deg_kernel(NCH, NA)
    edge_kernel = _make_edge_kernel(N, NCH, NA, H)

    deg_parts = deg_kernel(dstp).reshape(NT, NA)
    deg_t = deg_parts.T

    xw1 = _matmul1(x, W1, NB)
    y1, dis = _scale1(xw1, deg_t, NB)
    parts1 = edge_kernel(y1, srcp, dstp)
    y2 = _finish_matmul2(parts1, y1, dis, b1.reshape(1, H), W2, NB)
    parts2 = edge_kernel(y2, srcp, dstp)
    return _finish_pool_classify(parts2, y2, dis, b2.reshape(1, H),
                                 batch3, Wc, bc.reshape(1, Wc.shape[1]), NB)


# speedup vs baseline: 1.0544x; 1.0544x over previous
import functools

import jax
import jax.numpy as jnp
from jax import lax
from jax.experimental import pallas as pl
from jax.experimental.pallas import tpu as pltpu
from jax.experimental.pallas import tpu_sc as plsc

NT = 32
CH = 128
G = 64
BN = 5000

_sc_params = pltpu.CompilerParams(needs_layout_passes=False)
_mesh = plsc.VectorSubcoreMesh(core_axis_name="c", subcore_axis_name="s")


def _stripe_chunks(stripe, maxsz=CH):
    sizes = [maxsz] * (stripe // maxsz)
    if stripe % maxsz:
        sizes.append(stripe % maxsz)
    return sizes


def _make_deg_kernel(NCH, NA):
    @functools.partial(
        pl.kernel,
        out_type=jax.ShapeDtypeStruct((NT * NA,), jnp.float32),
        mesh=_mesh,
        scratch_types=[
            pltpu.VMEM((NCH, CH), jnp.int32),
            pltpu.VMEM((NA,), jnp.float32),
        ],
        compiler_params=_sc_params,
    )
    def deg_kernel(dst_hbm, out_hbm, dstv, degv):
        c = lax.axis_index("c")
        s = lax.axis_index("s")
        w = c * 16 + s
        pltpu.sync_copy(dst_hbm.at[w], dstv)

        zeros16 = jnp.zeros((16,), jnp.float32)

        def zero_body(i, carry):
            degv[pl.ds(i * 16, 16)] = zeros16
            return carry

        lax.fori_loop(0, NA // 16, zero_body, 0)

        ones16 = jnp.ones((16,), jnp.float32)

        def hist_body(r, carry):
            for k in range(CH // 16):
                idx = dstv[r, pl.ds(k * 16, 16)]
                plsc.addupdate_scatter(degv, [idx], ones16)
            return carry

        lax.fori_loop(0, NCH, hist_body, 0)
        pltpu.sync_copy(degv, out_hbm.at[pl.ds(w * NA, NA)])

    return deg_kernel


def _make_edge_kernel(N, NCH, NA, D):
    EP_T = NCH * CH
    stripe = NA // 16

    @functools.partial(
        pl.kernel,
        out_type=jax.ShapeDtypeStruct((2, NA, D), jnp.float32),
        mesh=_mesh,
        scratch_types=[
            pltpu.VMEM((EP_T,), jnp.int32),
            pltpu.VMEM((2, CH), jnp.int32),
            pltpu.VMEM((2, CH, D), jnp.float32),
            pltpu.VMEM_SHARED((NA, D), jnp.float32),
            pltpu.SemaphoreType.DMA,
            pltpu.SemaphoreType.DMA,
            pltpu.SemaphoreType.DMA,
            pltpu.SemaphoreType.DMA,
            pltpu.SemaphoreType.DMA,
            pltpu.SemaphoreType.DMA,
        ],
        compiler_params=_sc_params,
    )
    def edge_kernel(y_hbm, src_hbm, dst_hbm, out_hbm, srcv, dstb, buf, acc,
                    sem0, sem1, wsem0, wsem1, dsem0, dsem1):
        c = lax.axis_index("c")
        s = lax.axis_index("s")
        w = c * 16 + s
        pltpu.sync_copy(src_hbm.at[w], srcv)

        sems = (sem0, sem1)
        dsems = (dsem0, dsem1)

        def gather(j, slot, sem):
            return pltpu.async_copy(
                y_hbm.at[srcv.at[pl.ds(j * CH, CH)]], buf.at[slot], sem)

        def dst_load(j, slot):
            return pltpu.async_copy(dst_hbm.at[w, j], dstb.at[slot],
                                    dsems[slot])

        gather(0, 0, sem0)
        dst_load(0, 0)
        dst_load(1, 1)

        zeros16 = jnp.zeros((16,), jnp.float32)

        def zero_body(i, carry):
            r = i // (D // 16)
            k = lax.rem(i, D // 16)
            buf[1, r, pl.ds(k * 16, 16)] = zeros16
            return carry

        lax.fori_loop(0, CH * D // 16, zero_body, 0)

        base = s * stripe
        off = 0
        for sz in _stripe_chunks(stripe):
            pltpu.sync_copy(buf.at[1, pl.ds(0, sz)],
                            acc.at[pl.ds(base + off, sz)])
            off += sz
        plsc.subcore_barrier()

        def body(jj, carry):
            for b in range(2):
                j = jj * 2 + b
                nslot = 1 - b

                @pl.when(j + 1 < NCH)
                def _():
                    gather(j + 1, nslot, sems[nslot])

                pltpu.make_async_copy(
                    y_hbm.at[srcv.at[pl.ds(j * CH, CH)]], buf.at[b],
                    sems[b]).wait()
                pltpu.make_async_copy(dst_hbm.at[w, j], dstb.at[b],
                                      dsems[b]).wait()
                pltpu.sync_copy(buf.at[b], acc.at[dstb.at[b]], add=True)

                @pl.when(j + 2 < NCH)
                def _():
                    dst_load(j + 2, b)
            return carry

        lax.fori_loop(0, NCH // 2, body, 0)
        plsc.subcore_barrier()

        wsems = (wsem0, wsem1)
        chunks = []
        off = 0
        for sz in _stripe_chunks(stripe):
            chunks.append((off, sz))
            off += sz
        for k, (off, sz) in enumerate(chunks):
            slot = k % 2
            if k >= 2:
                poff, psz = chunks[k - 2]
                pltpu.make_async_copy(
                    buf.at[slot, pl.ds(0, psz)],
                    out_hbm.at[c, pl.ds(base + poff, psz)],
                    wsems[slot]).wait()
            pltpu.sync_copy(acc.at[pl.ds(base + off, sz)],
                            buf.at[slot, pl.ds(0, sz)])
            pltpu.async_copy(buf.at[slot, pl.ds(0, sz)],
                             out_hbm.at[c, pl.ds(base + off, sz)],
                             wsems[slot])
        for k in range(max(0, len(chunks) - 2), len(chunks)):
            off, sz = chunks[k]
            pltpu.make_async_copy(
                buf.at[k % 2, pl.ds(0, sz)],
                out_hbm.at[c, pl.ds(base + off, sz)],
                wsems[k % 2]).wait()

    return edge_kernel


def _matmul1(x, W1, NB):
    N, D = x.shape
    H = W1.shape[1]

    def body(x_ref, w_ref, xw_ref):
        xw_ref[...] = jnp.dot(x_ref[...], w_ref[...],
                              preferred_element_type=jnp.float32)

    return pl.pallas_call(
        body,
        grid=(NB,),
        in_specs=[
            pl.BlockSpec((BN, D), lambda i: (i, 0)),
            pl.BlockSpec((D, H), lambda i: (0, 0)),
        ],
        out_specs=pl.BlockSpec((BN, H), lambda i: (i, 0)),
        out_shape=jax.ShapeDtypeStruct((N, H), jnp.float32),
    )(x, W1)


def _scale1(xw, deg_t, NB):
    N, H = xw.shape

    def body(xw_ref, degt_ref, y_ref, dis_ref):
        deg = jnp.sum(degt_ref[...], axis=1, keepdims=True) + 1.0
        dis = lax.rsqrt(deg)
        y_ref[...] = xw_ref[...] * dis
        dis_ref[...] = dis

    return pl.pallas_call(
        body,
        grid=(NB,),
        in_specs=[
            pl.BlockSpec((BN, H), lambda i: (i, 0)),
            pl.BlockSpec((BN, NT), lambda i: (i, 0)),
        ],
        out_specs=[
            pl.BlockSpec((BN, H), lambda i: (i, 0)),
            pl.BlockSpec((BN, 1), lambda i: (i, 0)),
        ],
        out_shape=[
            jax.ShapeDtypeStruct((N, H), jnp.float32),
            jax.ShapeDtypeStruct((N, 1), jnp.float32),
        ],
    )(xw, deg_t)


def _finish_matmul2(parts, y1, dis, b1, W2, NB):
    N, H = y1.shape

    def body(p_ref, y_ref, dis_ref, b_ref, w_ref, out_ref):
        pre = p_ref[0] + p_ref[1] + y_ref[...]
        h = jnp.maximum(dis_ref[...] * pre + b_ref[...], 0.0)
        out_ref[...] = dis_ref[...] * jnp.dot(
            h, w_ref[...], preferred_element_type=jnp.float32)

    return pl.pallas_call(
        body,
        grid=(NB,),
        in_specs=[
            pl.BlockSpec((2, BN, H), lambda i: (0, i, 0)),
            pl.BlockSpec((BN, H), lambda i: (i, 0)),
            pl.BlockSpec((BN, 1), lambda i: (i, 0)),
            pl.BlockSpec((1, H), lambda i: (0, 0)),
            pl.BlockSpec((H, H), lambda i: (0, 0)),
        ],
        out_specs=pl.BlockSpec((BN, H), lambda i: (i, 0)),
        out_shape=jax.ShapeDtypeStruct((N, H), jnp.float32),
    )(parts, y1, dis, b1, W2)


def _finish_pool_classify(parts, y2, dis, b2, batch3, Wc, bc, NB):
    N, H = y2.shape
    C = Wc.shape[1]

    def body(p_ref, y_ref, dis_ref, b_ref, bat_ref, wc_ref, bc_ref, out_ref,
             pooled, cnt):
        i = pl.program_id(0)

        @pl.when(i == 0)
        def _():
            pooled[...] = jnp.zeros_like(pooled)
            cnt[...] = jnp.zeros_like(cnt)

        pre = p_ref[0] + p_ref[1] + y_ref[...]
        h = jnp.maximum(dis_ref[...] * pre + b_ref[...], 0.0)
        bat = bat_ref[0]
        gid = lax.broadcasted_iota(jnp.int32, (G, BN), 0)
        oh = (bat == gid).astype(jnp.float32)
        pooled[...] += jnp.dot(oh, h, preferred_element_type=jnp.float32)
        cnt[...] += jnp.broadcast_to(
            jnp.sum(oh, axis=1, keepdims=True), cnt.shape)

        @pl.when(i == NB - 1)
        def _():
            mean = pooled[...] / jnp.maximum(cnt[...], 1.0)
            out_ref[...] = jnp.dot(
                mean, wc_ref[...], preferred_element_type=jnp.float32) \
                + bc_ref[...]

    return pl.pallas_call(
        body,
        grid=(NB,),
        in_specs=[
            pl.BlockSpec((2, BN, H), lambda i: (0, i, 0)),
            pl.BlockSpec((BN, H), lambda i: (i, 0)),
            pl.BlockSpec((BN, 1), lambda i: (i, 0)),
            pl.BlockSpec((1, H), lambda i: (0, 0)),
            pl.BlockSpec((1, 1, BN), lambda i: (i, 0, 0)),
            pl.BlockSpec((H, C), lambda i: (0, 0)),
            pl.BlockSpec((1, C), lambda i: (0, 0)),
        ],
        out_specs=pl.BlockSpec((G, C), lambda i: (0, 0)),
        out_shape=jax.ShapeDtypeStruct((G, C), jnp.float32),
        scratch_shapes=[
            pltpu.VMEM((G, H), jnp.float32),
            pltpu.VMEM((G, H), jnp.float32),
        ],
    )(parts, y2, dis, b2, batch3, Wc, bc)


def kernel(x, edge_index, batch, W1, b1, W2, b2, Wc, bc):
    N, D = x.shape
    H = W1.shape[1]
    E = edge_index.shape[1]
    src, dst = edge_index[0], edge_index[1]

    NCH = -(-E // (NT * CH))
    NCH += NCH % 2
    EP = NT * NCH * CH
    NA = -(-(N + 16) // 128) * 128
    NB = N // BN

    npad = EP - E
    pad_src = (jnp.arange(npad, dtype=jnp.int32) * 97) % N
    pad_dst = N + jnp.arange(npad, dtype=jnp.int32) % 16
    srcp = jnp.concatenate([src, pad_src]).reshape(NT, NCH * CH)
    dstp = jnp.concatenate([dst, pad_dst]).reshape(NT, NCH, CH)
    batch3 = batch.reshape(NB, 1, BN)

    deg_kernel = _make_deg_kernel(NCH, NA)
    edge_kernel = _make_edge_kernel(N, NCH, NA, H)

    deg_parts = deg_kernel(dstp).reshape(NT, NA)
    deg_t = deg_parts.T

    xw1 = _matmul1(x, W1, NB)
    y1, dis = _scale1(xw1, deg_t, NB)
    parts1 = edge_kernel(y1, srcp, dstp)
    y2 = _finish_matmul2(parts1, y1, dis, b1.reshape(1, H), W2, NB)
    parts2 = edge_kernel(y2, srcp, dstp)
    return _finish_pool_classify(parts2, y2, dis, b2.reshape(1, H),
                                 batch3, Wc, bc.reshape(1, Wc.shape[1]), NB)
